# Initial kernel scaffold; baseline (speedup 1.0000x reference)
import jax
import jax.numpy as jnp
from jax.experimental import pallas as pl


def kernel(node_indices, edge_index, unpack_index, node_mask, embedding, msg_w, gru_wih, gru_whh, gru_bih, gru_bhh):
    raise NotImplementedError("write your pallas kernel here")


# speedup vs baseline: 2.0691x; 2.0691x over previous
import functools

import jax
import jax.numpy as jnp
from jax import lax
from jax.experimental import pallas as pl
from jax.experimental.pallas import tpu as pltpu
from jax.experimental.pallas import tpu_sc as plsc

N_NODES = 10000
N_EDGES = 160000
D = 128
NP = 10240
BLK = 512
GRID = NP // BLK
NW = 32
CPT = D // NW
CH = 1600
NCH = N_EDGES // CH

_HI = jax.lax.Precision.HIGHEST


def _wid():
    return lax.axis_index("s") * 2 + lax.axis_index("c")


def _embed_gather_body(emb_hbm, idx_hbm, out_hbm, idxv, rows, sem):
    w = _wid()
    pltpu.sync_copy(idx_hbm.at[w], idxv)
    for j in range(5):
        pltpu.async_copy(emb_hbm.at[idxv.at[j]], rows.at[pl.ds(j * 64, 64)], sem)
    for j in range(5):
        pltpu.make_async_copy(emb_hbm.at[idxv.at[0]], rows.at[pl.ds(0, 64)], sem).wait()
    pltpu.sync_copy(rows, out_hbm.at[pl.ds(w * 320, 320)])


def _edge_scatter_body(hw_hbm, src_hbm, dst_hbm, out_hbm, hwv, msgs,
                       sbuf0, sbuf1, dbuf0, dbuf1, sem0, sem1):
    w = _wid()
    sems = (sem0, sem1)
    sbufs = (sbuf0, sbuf1)
    dbufs = (dbuf0, dbuf1)
    pltpu.sync_copy(hw_hbm.at[0, w], hwv.at[pl.ds(0, CPT)])
    pltpu.sync_copy(hw_hbm.at[1, w], hwv.at[pl.ds(CPT, CPT)])

    zero16 = jnp.zeros((16,), jnp.float32)

    @pl.loop(0, NP // 16, unroll=8)
    def _(i):
        for p in range(CPT):
            msgs[p, pl.ds(i * 16, 16)] = zero16

    for t in range(2):
        pltpu.async_copy(src_hbm.at[pl.ds(0, CH)], sbuf0, sem0)
        pltpu.async_copy(dst_hbm.at[pl.ds(0, CH)], dbuf0, sem0)
        pltpu.async_copy(src_hbm.at[pl.ds(CH, CH)], sbuf1, sem1)
        pltpu.async_copy(dst_hbm.at[pl.ds(CH, CH)], dbuf1, sem1)

        @pl.loop(0, NCH // 2)
        def _(cc):
            for b in range(2):
                ch = cc * 2 + b
                pltpu.make_async_copy(src_hbm.at[pl.ds(0, CH)], sbufs[b], sems[b]).wait()
                pltpu.make_async_copy(dst_hbm.at[pl.ds(0, CH)], dbufs[b], sems[b]).wait()

                @pl.loop(0, CH // 16)
                def _(g):
                    base = g * 16
                    sv = sbufs[b][pl.ds(base, 16)]
                    dv = dbufs[b][pl.ds(base, 16)]
                    gv, cv = (sv, dv) if t == 0 else (dv, sv)
                    for p in range(CPT):
                        pv = jnp.full((16,), p, jnp.int32)
                        v = plsc.load_gather(hwv, [pv + t * CPT, gv])
                        plsc.addupdate_scatter(msgs, [pv, cv], v)

                @pl.when(ch + 2 < NCH)
                def _():
                    pltpu.async_copy(src_hbm.at[pl.ds((ch + 2) * CH, CH)], sbufs[b], sems[b])
                    pltpu.async_copy(dst_hbm.at[pl.ds((ch + 2) * CH, CH)], dbufs[b], sems[b])

    pltpu.sync_copy(msgs, out_hbm.at[w])


def _unpack_gather_body(h_hbm, idx_hbm, out_hbm, idxv, rows, sem):
    w = _wid()
    pltpu.sync_copy(idx_hbm.at[w], idxv)
    for j in range(4):
        pltpu.async_copy(h_hbm.at[idxv.at[j]], rows.at[pl.ds(j * 128, 128)], sem)
    for j in range(4):
        pltpu.make_async_copy(h_hbm.at[idxv.at[0]], rows.at[pl.ds(0, 128)], sem).wait()
    pltpu.sync_copy(rows, out_hbm.at[pl.ds(w * 512, 512)])


@functools.cache
def _sc_kernels():
    mesh = plsc.VectorSubcoreMesh(core_axis_name="c", subcore_axis_name="s")
    embed = pl.kernel(
        _embed_gather_body,
        out_type=jax.ShapeDtypeStruct((NP, D), jnp.float32),
        mesh=mesh,
        scratch_types=[
            pltpu.VMEM((5, 64), jnp.int32),
            pltpu.VMEM((320, D), jnp.float32),
            pltpu.SemaphoreType.DMA,
        ],
    )
    scatter = pl.kernel(
        _edge_scatter_body,
        out_type=jax.ShapeDtypeStruct((NW, CPT, NP), jnp.float32),
        mesh=mesh,
        compiler_params=pltpu.CompilerParams(needs_layout_passes=False),
        scratch_types=[
            pltpu.VMEM((2 * CPT, NP), jnp.float32),
            pltpu.VMEM((CPT, NP), jnp.float32),
            pltpu.VMEM((CH,), jnp.int32),
            pltpu.VMEM((CH,), jnp.int32),
            pltpu.VMEM((CH,), jnp.int32),
            pltpu.VMEM((CH,), jnp.int32),
            pltpu.SemaphoreType.DMA,
            pltpu.SemaphoreType.DMA,
        ],
    )
    unpack = pl.kernel(
        _unpack_gather_body,
        out_type=jax.ShapeDtypeStruct((16384, D), jnp.float32),
        mesh=mesh,
        scratch_types=[
            pltpu.VMEM((4, 128), jnp.int32),
            pltpu.VMEM((512, D), jnp.float32),
            pltpu.SemaphoreType.DMA,
        ],
    )
    return embed, scatter, unpack


def _dot(a, b):
    return jnp.dot(a, b, preferred_element_type=jnp.float32)


def _prep_body(h0_ref, mwT_ref, hT_ref, hwT_ref):
    hTb = h0_ref[...].T
    hT_ref[...] = hTb
    hwT_ref[0] = _dot(mwT_ref[0], hTb)
    hwT_ref[1] = _dot(mwT_ref[1], hTb)


_prep_call = pl.pallas_call(
    _prep_body,
    grid=(GRID,),
    in_specs=[
        pl.BlockSpec((BLK, D), lambda n: (n, 0)),
        pl.BlockSpec((2, D, D), lambda n: (0, 0, 0)),
    ],
    out_specs=[
        pl.BlockSpec((D, BLK), lambda n: (0, n)),
        pl.BlockSpec((2, D, BLK), lambda n: (0, 0, n)),
    ],
    out_shape=[
        jax.ShapeDtypeStruct((D, NP), jnp.float32),
        jax.ShapeDtypeStruct((2, D, NP), jnp.float32),
    ],
)


def _gru_body(final, msgsT_ref, hT_ref, wih_ref, whh_ref, bih_ref, bhh_ref, mwT_ref,
              out0_ref, out1_ref=None):
    m = msgsT_ref[...]
    h = hT_ref[...]
    gi = _dot(wih_ref[...], m) + bih_ref[...]
    gh = _dot(whh_ref[...], h) + bhh_ref[...]
    r = jax.nn.sigmoid(gi[0:D] + gh[0:D])
    z = jax.nn.sigmoid(gi[D:2 * D] + gh[D:2 * D])
    n = jnp.tanh(gi[2 * D:3 * D] + r * gh[2 * D:3 * D])
    hnew = (1.0 - z) * n + z * h
    if final:
        out0_ref[...] = hnew.T
    else:
        out0_ref[...] = hnew
        out1_ref[0] = _dot(mwT_ref[0], hnew)
        out1_ref[1] = _dot(mwT_ref[1], hnew)


_gru_in_specs = [
    pl.BlockSpec((D, BLK), lambda n: (0, n)),
    pl.BlockSpec((D, BLK), lambda n: (0, n)),
    pl.BlockSpec((3 * D, D), lambda n: (0, 0)),
    pl.BlockSpec((3 * D, D), lambda n: (0, 0)),
    pl.BlockSpec((3 * D, BLK), lambda n: (0, 0)),
    pl.BlockSpec((3 * D, BLK), lambda n: (0, 0)),
    pl.BlockSpec((2, D, D), lambda n: (0, 0, 0)),
]

_gru_call = pl.pallas_call(
    functools.partial(_gru_body, False),
    grid=(GRID,),
    in_specs=_gru_in_specs,
    out_specs=[
        pl.BlockSpec((D, BLK), lambda n: (0, n)),
        pl.BlockSpec((2, D, BLK), lambda n: (0, 0, n)),
    ],
    out_shape=[
        jax.ShapeDtypeStruct((D, NP), jnp.float32),
        jax.ShapeDtypeStruct((2, D, NP), jnp.float32),
    ],
)

_gru_final_call = pl.pallas_call(
    functools.partial(_gru_body, True),
    grid=(GRID,),
    in_specs=_gru_in_specs,
    out_specs=[pl.BlockSpec((BLK, D), lambda n: (n, 0))],
    out_shape=[jax.ShapeDtypeStruct((NP, D), jnp.float32)],
)


def kernel(node_indices, edge_index, unpack_index, node_mask, embedding,
           msg_w, gru_wih, gru_whh, gru_bih, gru_bhh):
    nidx = jnp.concatenate(
        [node_indices.astype(jnp.int32),
         jnp.zeros((NP - N_NODES,), jnp.int32)]).reshape(NW, 5, 64)
    src = edge_index[0].astype(jnp.int32)
    dst = edge_index[1].astype(jnp.int32)
    msg_wT = jnp.swapaxes(msg_w, -1, -2)
    bihb = jnp.broadcast_to(gru_bih[:, :, None], (2, 3 * D, BLK)).astype(jnp.float32)
    bhhb = jnp.broadcast_to(gru_bhh[:, :, None], (2, 3 * D, BLK)).astype(jnp.float32)

    _embed_gather, _edge_scatter, _unpack_gather = _sc_kernels()
    h0 = _embed_gather(embedding, nidx)
    hT, hwT = _prep_call(h0, msg_wT[0])

    hrows = None
    for step in range(10):
        layer = step // 5
        msgsT = _edge_scatter(hwT.reshape(2, NW, CPT, NP), src, dst)
        msgsT = msgsT.reshape(D, NP)
        if step < 9:
            nl = (step + 1) // 5
            hT, hwT = _gru_call(msgsT, hT, gru_wih[layer], gru_whh[layer],
                                bihb[layer], bhhb[layer], msg_wT[nl])
        else:
            (hrows,) = _gru_final_call(msgsT, hT, gru_wih[layer], gru_whh[layer],
                                       bihb[layer], bhhb[layer], msg_wT[layer])

    uidx = unpack_index.reshape(-1).astype(jnp.int32).reshape(NW, 4, 128)
    enc = _unpack_gather(hrows, uidx)
    return enc.reshape(16, 1024, D) * node_mask[..., None]
